# baseline (speedup 1.0000x reference)
import jax
import jax.numpy as jnp
from jax.experimental import pallas as pl
from jax.experimental.pallas import tpu as pltpu

EPS = 1e-7
K = 2048


def _matmul_kernel(x_ref, w_ref, o_ref):
    o_ref[...] = jnp.dot(x_ref[...], w_ref[...],
                         preferred_element_type=jnp.float32)


def _matmul(x, w, blk):
    n = x.shape[0]
    return pl.pallas_call(
        _matmul_kernel,
        grid=(n // blk,),
        in_specs=[pl.BlockSpec((blk, x.shape[1]), lambda i: (i, 0)),
                  pl.BlockSpec(w.shape, lambda i: (0, 0))],
        out_specs=pl.BlockSpec((blk, w.shape[1]), lambda i: (i, 0)),
        out_shape=jax.ShapeDtypeStruct((n, w.shape[1]), jnp.float32),
    )(x, w)


def _segred_kernel(T, eb_ref, ot_ref, fv_ref, vd_ref,
                   seg_ref, scl_ref, val_ref, out_ref):
    p = pl.program_id(0)
    ot = ot_ref[p]

    @pl.when(fv_ref[p] == 1)
    def _():
        out_ref[...] = jnp.zeros_like(out_ref)

    @pl.when(vd_ref[p] == 1)
    def _():
        seg = seg_ref[0, 0, :]
        scl = scl_ref[0, 0, :]
        local = seg - ot * T
        oh = (local[:, None] ==
              jax.lax.broadcasted_iota(jnp.int32, (K, T), 1)
              ).astype(jnp.float32)
        scaled = val_ref[...] * scl[:, None]
        out_ref[:, :64] += jax.lax.dot_general(
            oh, scaled, (((0,), (0,)), ((), ())),
            preferred_element_type=jnp.float32)
        out_ref[:, 64:65] += jax.lax.dot_general(
            oh, scl[:, None], (((0,), (0,)), ((), ())),
            preferred_element_type=jnp.float32)


def _segment_reduce(seg_sorted, scale, values, n_out_pad, T):
    e_pad = seg_sorted.shape[0]
    nb = e_pad // K
    nt = n_out_pad // T
    segs = seg_sorted.reshape(nb, K)
    f = segs[:, 0] // T
    l = segs[:, -1] // T
    f = jnp.concatenate([jnp.zeros((1,), jnp.int32),
                         jnp.minimum(f[1:], l[:-1] + 1)])
    l = l.at[-1].set(nt - 1)
    spans = l - f + 1
    offs = jnp.concatenate([jnp.zeros((1,), jnp.int32),
                            jnp.cumsum(spans)])
    P = nb + nt
    parr = jnp.arange(P, dtype=jnp.int32)
    b = jnp.clip(jnp.searchsorted(offs, parr, side='right') - 1, 0, nb - 1)
    ot = jnp.clip(f[b] + (parr - offs[b]), 0, nt - 1)
    valid = (parr < offs[-1]).astype(jnp.int32)
    fv = jnp.concatenate([jnp.ones((1,), jnp.int32),
                          (ot[1:] != ot[:-1]).astype(jnp.int32)])
    eb = b

    import functools
    grid_spec = pltpu.PrefetchScalarGridSpec(
        num_scalar_prefetch=4,
        grid=(P,),
        in_specs=[
            pl.BlockSpec((1, 1, K), lambda p, eb, ot, fv, vd: (eb[p], 0, 0)),
            pl.BlockSpec((1, 1, K), lambda p, eb, ot, fv, vd: (eb[p], 0, 0)),
            pl.BlockSpec((K, 64), lambda p, eb, ot, fv, vd: (eb[p], 0)),
        ],
        out_specs=pl.BlockSpec((T, 128), lambda p, eb, ot, fv, vd: (ot[p], 0)),
    )
    return pl.pallas_call(
        functools.partial(_segred_kernel, T),
        grid_spec=grid_spec,
        out_shape=jax.ShapeDtypeStruct((n_out_pad, 128), jnp.float32),
    )(eb, ot, fv, valid,
      seg_sorted.reshape(nb, 1, K), scale.reshape(nb, 1, K),
      values)


def _combine_kernel(h_ref, ad_ref, o_ref):
    agg = ad_ref[:, :64]
    deg = ad_ref[:, 64:65]
    o_ref[...] = jax.nn.relu(h_ref[...] + agg / (deg + EPS))


def _divide_kernel(sc_ref, o_ref):
    s = sc_ref[:, :64]
    c = sc_ref[:, 64:65]
    o_ref[...] = s / jnp.maximum(c, 1.0)


def _region_kernel(rs_ref, rsrc_ref, rdst_ref, w_ref, area_ref, perm_ref,
                   pos_ref, neg_ref, city_ref):
    r0 = rs_ref[:, :64] / jnp.maximum(rs_ref[:, 64:65], 1.0)
    rsrc = rsrc_ref[0, :]
    rdst = rdst_ref[0, :]
    ncols = r0.shape[0]
    iota = jax.lax.broadcasted_iota(jnp.int32, (rsrc.shape[0], ncols), 1)
    oh_src = (rsrc[:, None] == iota).astype(jnp.float32)
    oh_dst = (rdst[:, None] == iota).astype(jnp.float32)
    gathered = jnp.dot(oh_src, r0, preferred_element_type=jnp.float32)
    nmsg = jax.lax.dot_general(oh_dst, gathered, (((0,), (0,)), ((), ())),
                               preferred_element_type=jnp.float32)
    ndeg = jnp.sum(oh_dst, axis=0)[:, None]
    pos = jnp.tanh(jnp.dot(r0 + nmsg / jnp.maximum(ndeg, 1.0), w_ref[...],
                           preferred_element_type=jnp.float32))
    pos_ref[...] = pos
    perm = perm_ref[0, :]
    iota2 = jax.lax.broadcasted_iota(jnp.int32, (ncols, ncols), 1)
    permoh = (perm[:, None] == iota2).astype(jnp.float32)
    neg_ref[...] = jnp.dot(permoh, pos, preferred_element_type=jnp.float32)
    area = area_ref[...]
    w = area / (jnp.sum(area) + EPS)
    city_ref[...] = jax.nn.sigmoid(
        jnp.dot(w, pos, preferred_element_type=jnp.float32))


def kernel(x, edge_weight, region_area, W_enc, W_region, edge_index,
           checkin_to_poi, poi_to_region, region_adjacency,
           num_pois, num_regions):
    n = x.shape[0]
    num_pois_static = poi_to_region.shape[0]
    num_regions_static = region_area.shape[0]
    src, dst = edge_index[0], edge_index[1]
    n_edges = src.shape[0]

    h = _matmul(x, W_enc, 2000)

    order = jnp.argsort(dst)
    e_pad = ((n_edges + K - 1) // K) * K
    n_pad = ((n + K - 1) // K) * K
    pad_e = e_pad - n_edges
    dst_s = jnp.concatenate([dst[order],
                             jnp.full((pad_e,), n_pad - 1, dst.dtype)])
    src_s = jnp.concatenate([src[order], jnp.zeros((pad_e,), src.dtype)])
    w_s = jnp.concatenate([edge_weight[order],
                           jnp.zeros((pad_e,), jnp.float32)])
    h_src = jnp.take(h, src_s, axis=0)
    aggdeg = _segment_reduce(dst_s, w_s, h_src, n_pad, 256)[:n]

    pos_checkin_emb = pl.pallas_call(
        _combine_kernel,
        grid=(n // 2000,),
        in_specs=[pl.BlockSpec((2000, 64), lambda i: (i, 0)),
                  pl.BlockSpec((2000, 128), lambda i: (i, 0))],
        out_specs=pl.BlockSpec((2000, 64), lambda i: (i, 0)),
        out_shape=jax.ShapeDtypeStruct((n, 64), jnp.float32),
    )(h, aggdeg)

    c_pad = ((n + K - 1) // K) * K
    p_out_pad = ((num_pois_static + 255) // 256) * 256
    padc = c_pad - n
    c2p = jnp.concatenate([checkin_to_poi,
                           jnp.full((padc,), p_out_pad - 1,
                                    checkin_to_poi.dtype)])
    ones_c = jnp.concatenate([jnp.ones((n,), jnp.float32),
                              jnp.zeros((padc,), jnp.float32)])
    vals_c = jnp.concatenate([pos_checkin_emb,
                              jnp.zeros((padc, 64), jnp.float32)])
    poi_sc = _segment_reduce(c2p, ones_c, vals_c, p_out_pad, 256)
    pos_poi_emb = pl.pallas_call(
        _divide_kernel,
        grid=(p_out_pad // 2048,),
        in_specs=[pl.BlockSpec((2048, 128), lambda i: (i, 0))],
        out_specs=pl.BlockSpec((2048, 64), lambda i: (i, 0)),
        out_shape=jax.ShapeDtypeStruct((p_out_pad, 64), jnp.float32),
    )(poi_sc)[:num_pois_static]

    p_pad = ((num_pois_static + K - 1) // K) * K
    padp = p_pad - num_pois_static
    p2r = jnp.concatenate([poi_to_region,
                           jnp.full((padp,), num_regions_static - 1,
                                    poi_to_region.dtype)])
    ones_p = jnp.concatenate([jnp.ones((num_pois_static,), jnp.float32),
                              jnp.zeros((padp,), jnp.float32)])
    vals_p = jnp.concatenate([pos_poi_emb,
                              jnp.zeros((padp, 64), jnp.float32)])
    reg_sc = _segment_reduce(p2r, ones_p, vals_p, num_regions_static,
                             num_regions_static)

    perm = jax.random.permutation(jax.random.key(1), num_regions_static)
    pos_region_emb, neg_region_emb, city2 = pl.pallas_call(
        _region_kernel,
        out_shape=[
            jax.ShapeDtypeStruct((num_regions_static, 64), jnp.float32),
            jax.ShapeDtypeStruct((num_regions_static, 64), jnp.float32),
            jax.ShapeDtypeStruct((1, 64), jnp.float32),
        ],
    )(reg_sc,
      region_adjacency[0][None, :].astype(jnp.int32),
      region_adjacency[1][None, :].astype(jnp.int32),
      W_region,
      region_area[None, :],
      perm[None, :].astype(jnp.int32))
    city_emb = city2[0]

    pos_poi_expanded = jnp.take(pos_poi_emb, checkin_to_poi, axis=0)
    neg_poi_idx = jax.random.randint(jax.random.key(2), (n,), 0,
                                     num_pois - 1)
    neg_poi_idx = jnp.where(neg_poi_idx >= checkin_to_poi,
                            neg_poi_idx + 1, neg_poi_idx)
    neg_poi_expanded = jnp.take(pos_poi_emb, neg_poi_idx, axis=0)
    pos_region_expanded = jnp.take(pos_region_emb, poi_to_region, axis=0)
    neg_reg_idx = jax.random.randint(jax.random.key(3),
                                     (num_pois_static,), 0,
                                     num_regions - 1)
    neg_reg_idx = jnp.where(neg_reg_idx >= poi_to_region,
                            neg_reg_idx + 1, neg_reg_idx)
    neg_region_expanded = jnp.take(pos_region_emb, neg_reg_idx, axis=0)

    return (pos_checkin_emb, pos_poi_expanded, neg_poi_expanded,
            pos_poi_emb, pos_region_expanded, neg_region_expanded,
            pos_region_emb, neg_region_emb, city_emb)
